# baseline (speedup 1.0000x reference)
import functools

import jax
import jax.numpy as jnp
from jax import lax
from jax.experimental import pallas as pl
from jax.experimental.pallas import tpu as pltpu
from jax.experimental.pallas import tpu_sc as plsc

_CYC = 168
_LEN = 512
_B = 1024
_D = 128
_EXT = 680

_info = plsc.get_sparse_core_info()
_NC = _info.num_cores
_NS = _info.num_subcores
_NW = _NC * _NS
_BPW = _B // _NW
_N_SPMEM = 14


def _stage_ext(dst, data_hbm, sem):
    stage = []
    for k in range(_EXT // _CYC):
        stage.append(pltpu.make_async_copy(
            data_hbm, dst.at[pl.ds(k * _CYC, _CYC)], sem))
    tail = _EXT - (_EXT // _CYC) * _CYC
    stage.append(pltpu.make_async_copy(
        data_hbm.at[pl.ds(0, tail)], dst.at[pl.ds(_EXT - tail, tail)], sem))
    for cp in stage:
        cp.start()
    return stage


@functools.partial(
    pl.kernel,
    out_type=jax.ShapeDtypeStruct((_B * _LEN, _D), jnp.float32),
    mesh=plsc.VectorSubcoreMesh(core_axis_name="c", subcore_axis_name="s",
                                num_cores=_NC),
    scratch_types=[
        pltpu.VMEM((_EXT, _D), jnp.float32),
        pltpu.VMEM_SHARED((_EXT, _D), jnp.float32),
        pltpu.VMEM((_BPW,), jnp.int32),
        pltpu.SemaphoreType.DMA,
        pltpu.SemaphoreType.DMA,
        pltpu.SemaphoreType.DMA,
    ],
)
def _cyclic_gather(idx_hbm, data_hbm, out_hbm, ext_v, ext_s, idx_v,
                   sem_t, sem_s, stage_sem):
    sid = lax.axis_index("s")
    wid = sid * _NC + lax.axis_index("c")
    base = wid * _BPW

    idx_cp = pltpu.make_async_copy(idx_hbm.at[pl.ds(base, _BPW)], idx_v,
                                   stage_sem)
    idx_cp.start()
    stage = _stage_ext(ext_v, data_hbm, stage_sem)

    ext_s_stage = []

    @pl.when(sid == 0)
    def _():
        ext_s_stage.extend(_stage_ext(ext_s, data_hbm, stage_sem))

    idx_cp.wait()
    for cp in stage:
        cp.wait()

    copies = []
    rows = []
    for g in range(_BPW // 16):
        vec = idx_v[pl.ds(g * 16, 16)] % _CYC
        for i in range(16):
            rows.append(vec[i])
    for i in range(_N_SPMEM, _BPW):
        cp = pltpu.make_async_copy(
            ext_v.at[pl.ds(rows[i], _LEN)],
            out_hbm.at[pl.ds((base + i) * _LEN, _LEN)], sem_t)
        cp.start()
        copies.append(cp)

    @pl.when(sid == 0)
    def _():
        for cp in ext_s_stage:
            cp.wait()

    plsc.subcore_barrier()
    for i in range(_N_SPMEM):
        cp = pltpu.make_async_copy(
            ext_s.at[pl.ds(rows[i], _LEN)],
            out_hbm.at[pl.ds((base + i) * _LEN, _LEN)], sem_s)
        cp.start()
        copies.append(cp)
    for cp in copies:
        cp.wait()


def kernel(index, length, data):
    del length
    out = _cyclic_gather(index.astype(jnp.int32), data)
    return out.reshape(_B, _LEN, _D)


# speedup vs baseline: 1.0139x; 1.0139x over previous
import functools

import jax
import jax.numpy as jnp
from jax import lax
from jax.experimental import pallas as pl
from jax.experimental.pallas import tpu as pltpu
from jax.experimental.pallas import tpu_sc as plsc

_CYC = 168
_LEN = 512
_B = 1024
_D = 128
_EXT = 680

_info = plsc.get_sparse_core_info()
_NC = _info.num_cores
_NS = _info.num_subcores
_NW = _NC * _NS
_BPW = _B // _NW
_N_SPMEM = 14


def _stage_ext(dst, data_hbm, sem):
    stage = []
    for k in range(_EXT // _CYC):
        stage.append(pltpu.make_async_copy(
            data_hbm, dst.at[pl.ds(k * _CYC, _CYC)], sem))
    tail = _EXT - (_EXT // _CYC) * _CYC
    stage.append(pltpu.make_async_copy(
        data_hbm.at[pl.ds(0, tail)], dst.at[pl.ds(_EXT - tail, tail)], sem))
    for cp in stage:
        cp.start()
    return stage


@functools.partial(
    pl.kernel,
    out_type=jax.ShapeDtypeStruct((_B * _LEN, _D), jnp.float32),
    mesh=plsc.VectorSubcoreMesh(core_axis_name="c", subcore_axis_name="s",
                                num_cores=_NC),
    scratch_types=[
        pltpu.VMEM((_EXT, _D), jnp.float32),
        pltpu.VMEM_SHARED((_EXT, _D), jnp.float32),
        pltpu.VMEM((_BPW,), jnp.int32),
        pltpu.SemaphoreType.DMA,
        pltpu.SemaphoreType.DMA,
        pltpu.SemaphoreType.DMA,
    ],
)
def _cyclic_gather(idx_hbm, data_hbm, out_hbm, ext_v, ext_s, idx_v,
                   sem_t, sem_s, stage_sem):
    sid = lax.axis_index("s")
    wid = sid * _NC + lax.axis_index("c")
    base = wid * _BPW

    idx_cp = pltpu.make_async_copy(idx_hbm.at[pl.ds(base, _BPW)], idx_v,
                                   stage_sem)
    idx_cp.start()
    stage = _stage_ext(ext_v, data_hbm, stage_sem)

    ext_s_stage = []

    @pl.when(sid == 0)
    def _():
        ext_s_stage.extend(_stage_ext(ext_s, data_hbm, stage_sem))

    @pl.when(sid == 0)
    def _():
        for cp in ext_s_stage:
            cp.wait()

    idx_cp.wait()
    for cp in stage:
        cp.wait()
    plsc.subcore_barrier()

    rows = []
    for g in range(_BPW // 16):
        vec = idx_v[pl.ds(g * 16, 16)] % _CYC
        for i in range(16):
            rows.append(vec[i])
    copies = []
    for i in range(_BPW):
        src = ext_s if i < _N_SPMEM else ext_v
        sem = sem_s if i < _N_SPMEM else sem_t
        cp = pltpu.make_async_copy(
            src.at[pl.ds(rows[i], _LEN)],
            out_hbm.at[pl.ds((base + i) * _LEN, _LEN)], sem)
        cp.start()
        copies.append(cp)
    for cp in copies:
        cp.wait()


def kernel(index, length, data):
    del length
    out = _cyclic_gather(index.astype(jnp.int32), data)
    return out.reshape(_B, _LEN, _D)
